# baseline (device time: 17475 ns/iter reference)
import jax
import jax.numpy as jnp
from jax import lax
from jax.experimental import pallas as pl
from jax.experimental.pallas import tpu as pltpu

C = 4
SCALE_ROWS = 8


def kernel(x):
    m, n = x.shape
    rows = m // C

    def body(
        x_hbm, out_hbm, xv, obuf, qsend, qrecv, ssend, srecv,
        in_sems, out_sems, ss_send, ss_recv, d_send, d_recv,
    ):
        my_x = lax.axis_index("x")
        my_y = lax.axis_index("y")
        my_z = lax.axis_index("z")
        z_peer = (my_x, my_y, 1 - my_z)

        in_dmas = []
        for c in range(C):
            sl = pl.ds(c * rows, rows)
            d = pltpu.make_async_copy(x_hbm.at[sl], xv.at[sl], in_sems.at[c])
            d.start()
            in_dmas.append(d)

        s_rdmas = []
        d_rdmas = []
        for c in range(C):
            sl = pl.ds(c * rows, rows)
            in_dmas[c].wait()
            s = jnp.max(jnp.abs(xv[sl, :])) / 127.0
            s = jnp.maximum(s, 1e-30)
            ssend[c, :] = jnp.full((n,), s, jnp.float32)
            inv = 1.0 / s
            qsend[sl, :] = jnp.clip(
                jnp.round(xv[sl, :] * inv), -127.0, 127.0
            ).astype(jnp.int8)
            sr = pltpu.make_async_remote_copy(
                src_ref=ssend.at[pl.ds(c, 1)],
                dst_ref=srecv.at[pl.ds(c, 1)],
                send_sem=ss_send.at[c],
                recv_sem=ss_recv.at[c],
                device_id=z_peer,
                device_id_type=pl.DeviceIdType.MESH,
            )
            sr.start()
            dr = pltpu.make_async_remote_copy(
                src_ref=qsend.at[sl],
                dst_ref=qrecv.at[sl],
                send_sem=d_send.at[c],
                recv_sem=d_recv.at[c],
                device_id=z_peer,
                device_id_type=pl.DeviceIdType.MESH,
            )
            dr.start()
            s_rdmas.append(sr)
            d_rdmas.append(dr)

        out_dmas = []
        for c in range(C):
            sl = pl.ds(c * rows, rows)
            s_rdmas[c].wait_recv()
            d_rdmas[c].wait_recv()
            peer_scale = srecv[c, 0]
            obuf[sl, :] = (
                xv[sl, :] + qrecv[sl, :].astype(jnp.float32) * peer_scale
            ).astype(jnp.bfloat16)
            d = pltpu.make_async_copy(obuf.at[sl], out_hbm.at[sl], out_sems.at[c])
            d.start()
            out_dmas.append(d)

        for c in range(C):
            out_dmas[c].wait()
            s_rdmas[c].wait_send()
            d_rdmas[c].wait_send()

    return pl.pallas_call(
        body,
        out_shape=jax.ShapeDtypeStruct((m, n), jnp.bfloat16),
        in_specs=[pl.BlockSpec(memory_space=pl.ANY)],
        out_specs=pl.BlockSpec(memory_space=pl.ANY),
        scratch_shapes=[
            pltpu.VMEM((m, n), jnp.float32),
            pltpu.VMEM((m, n), jnp.bfloat16),
            pltpu.VMEM((m, n), jnp.int8),
            pltpu.VMEM((m, n), jnp.int8),
            pltpu.VMEM((SCALE_ROWS, n), jnp.float32),
            pltpu.VMEM((SCALE_ROWS, n), jnp.float32),
            pltpu.SemaphoreType.DMA((C,)),
            pltpu.SemaphoreType.DMA((C,)),
            pltpu.SemaphoreType.DMA((C,)),
            pltpu.SemaphoreType.DMA((C,)),
            pltpu.SemaphoreType.DMA((C,)),
            pltpu.SemaphoreType.DMA((C,)),
        ],
    )(x)


# device time: 13405 ns/iter; 1.3036x vs baseline; 1.3036x over previous
import jax
import jax.numpy as jnp
from jax import lax
from jax.experimental import pallas as pl
from jax.experimental.pallas import tpu as pltpu

ROWS = (256, 256, 256, 192, 64)
C = len(ROWS)
OFFS = tuple(sum(ROWS[:i]) for i in range(C))
SCALE_ROWS = 8


def kernel(x):
    m, n = x.shape

    def body(
        x_hbm, out_hbm, xv, obuf, qsend, qrecv, ssend, srecv,
        in_sems, out_sems, ss_send, ss_recv, d_send, d_recv,
    ):
        my_x = lax.axis_index("x")
        my_y = lax.axis_index("y")
        my_z = lax.axis_index("z")
        z_peer = (my_x, my_y, 1 - my_z)

        in_dmas = []
        for c in range(C):
            sl = pl.ds(OFFS[c], ROWS[c])
            d = pltpu.make_async_copy(x_hbm.at[sl], xv.at[sl], in_sems.at[c])
            d.start()
            in_dmas.append(d)

        barrier_sem = pltpu.get_barrier_semaphore()
        pl.semaphore_signal(
            barrier_sem, inc=1, device_id=z_peer,
            device_id_type=pl.DeviceIdType.MESH,
        )

        def quantize(c):
            sl = pl.ds(OFFS[c], ROWS[c])
            in_dmas[c].wait()
            s = jnp.max(jnp.abs(xv[sl, :])) / 127.0
            s = jnp.maximum(s, 1e-30)
            ssend[c, :] = jnp.full((n,), s, jnp.float32)
            qsend[sl, :] = jnp.clip(
                jnp.round(xv[sl, :] * (1.0 / s)), -127.0, 127.0
            ).astype(jnp.int8)

        def send(c):
            sl = pl.ds(OFFS[c], ROWS[c])
            sr = pltpu.make_async_remote_copy(
                src_ref=ssend.at[pl.ds(c, 1)],
                dst_ref=srecv.at[pl.ds(c, 1)],
                send_sem=ss_send.at[c],
                recv_sem=ss_recv.at[c],
                device_id=z_peer,
                device_id_type=pl.DeviceIdType.MESH,
            )
            sr.start()
            dr = pltpu.make_async_remote_copy(
                src_ref=qsend.at[sl],
                dst_ref=qrecv.at[sl],
                send_sem=d_send.at[c],
                recv_sem=d_recv.at[c],
                device_id=z_peer,
                device_id_type=pl.DeviceIdType.MESH,
            )
            dr.start()
            return sr, dr

        quantize(0)
        pl.semaphore_wait(barrier_sem, 1)
        rdmas = [send(0)]
        for c in range(1, C):
            quantize(c)
            rdmas.append(send(c))

        out_dmas = []
        for c in range(C):
            sl = pl.ds(OFFS[c], ROWS[c])
            sr, dr = rdmas[c]
            sr.wait_recv()
            dr.wait_recv()
            peer_scale = srecv[c, 0]
            obuf[sl, :] = (
                xv[sl, :] + qrecv[sl, :].astype(jnp.float32) * peer_scale
            ).astype(jnp.bfloat16)
            d = pltpu.make_async_copy(obuf.at[sl], out_hbm.at[sl], out_sems.at[c])
            d.start()
            out_dmas.append(d)

        for c in range(C):
            out_dmas[c].wait()
            rdmas[c][0].wait_send()
            rdmas[c][1].wait_send()

    return pl.pallas_call(
        body,
        out_shape=jax.ShapeDtypeStruct((m, n), jnp.bfloat16),
        in_specs=[pl.BlockSpec(memory_space=pl.ANY)],
        out_specs=pl.BlockSpec(memory_space=pl.ANY),
        scratch_shapes=[
            pltpu.VMEM((m, n), jnp.float32),
            pltpu.VMEM((m, n), jnp.bfloat16),
            pltpu.VMEM((m, n), jnp.int8),
            pltpu.VMEM((m, n), jnp.int8),
            pltpu.VMEM((SCALE_ROWS, n), jnp.float32),
            pltpu.VMEM((SCALE_ROWS, n), jnp.float32),
            pltpu.SemaphoreType.DMA((C,)),
            pltpu.SemaphoreType.DMA((C,)),
            pltpu.SemaphoreType.DMA((C,)),
            pltpu.SemaphoreType.DMA((C,)),
            pltpu.SemaphoreType.DMA((C,)),
            pltpu.SemaphoreType.DMA((C,)),
        ],
        compiler_params=pltpu.CompilerParams(collective_id=0),
    )(x)


# device time: 13363 ns/iter; 1.3077x vs baseline; 1.0031x over previous
import jax
import jax.numpy as jnp
from jax import lax
from jax.experimental import pallas as pl
from jax.experimental.pallas import tpu as pltpu

ROWS = (256, 256, 256, 192, 64)
C = len(ROWS)
OFFS = tuple(sum(ROWS[:i]) for i in range(C))
SCALE_ROWS = 8


def kernel(x):
    m, n = x.shape

    def body(
        x_hbm, out_hbm, xv, obuf, qsend, qrecv, ssend, srecv,
        in_sems, out_sems, ss_send, ss_recv, d_send, d_recv,
    ):
        my_x = lax.axis_index("x")
        my_y = lax.axis_index("y")
        my_z = lax.axis_index("z")
        z_peer = (my_x, my_y, 1 - my_z)

        in_dmas = []
        for c in range(C):
            sl = pl.ds(OFFS[c], ROWS[c])
            d = pltpu.make_async_copy(x_hbm.at[sl], xv.at[sl], in_sems.at[c])
            d.start()
            in_dmas.append(d)

        barrier_sem = pltpu.get_barrier_semaphore()
        pl.semaphore_signal(
            barrier_sem, inc=1, device_id=z_peer,
            device_id_type=pl.DeviceIdType.MESH,
        )

        def quantize(c):
            sl = pl.ds(OFFS[c], ROWS[c])
            in_dmas[c].wait()
            s = jnp.max(jnp.abs(xv[sl, :])) / 127.0
            s = jnp.maximum(s, 1e-30)
            ssend[c, :] = jnp.full((n,), s, jnp.float32)
            qsend[sl, :] = jnp.clip(
                jnp.round(xv[sl, :] * (1.0 / s)), -127.0, 127.0
            ).astype(jnp.int8)

        def send(c):
            sl = pl.ds(OFFS[c], ROWS[c])
            sr = pltpu.make_async_remote_copy(
                src_ref=ssend.at[pl.ds(c, 1)],
                dst_ref=srecv.at[pl.ds(c, 1)],
                send_sem=ss_send.at[c],
                recv_sem=ss_recv.at[c],
                device_id=z_peer,
                device_id_type=pl.DeviceIdType.MESH,
            )
            sr.start()
            dr = pltpu.make_async_remote_copy(
                src_ref=qsend.at[sl],
                dst_ref=qrecv.at[sl],
                send_sem=d_send.at[c],
                recv_sem=d_recv.at[c],
                device_id=z_peer,
                device_id_type=pl.DeviceIdType.MESH,
            )
            dr.start()
            return sr, dr

        quantize(0)
        pl.semaphore_wait(barrier_sem, 1)
        rdmas = [send(0)]
        for c in range(1, C):
            quantize(c)
            rdmas.append(send(c))

        out_dmas = []
        for c in range(C):
            sl = pl.ds(OFFS[c], ROWS[c])
            sr, dr = rdmas[c]
            sr.wait_recv()
            dr.wait_recv()
            peer_scale = srecv[c, 0]
            obuf[sl, :] = (
                xv[sl, :] + qrecv[sl, :].astype(jnp.float32) * peer_scale
            ).astype(jnp.bfloat16)
            d = pltpu.make_async_copy(obuf.at[sl], out_hbm.at[sl], out_sems.at[c])
            d.start()
            out_dmas.append(d)

        for c in range(C):
            out_dmas[c].wait()
            rdmas[c][0].wait_send()
            rdmas[c][1].wait_send()

    return pl.pallas_call(
        body,
        out_shape=jax.ShapeDtypeStruct((m, n), jnp.bfloat16),
        in_specs=[pl.BlockSpec(memory_space=pltpu.MemorySpace.HBM)],
        out_specs=pl.BlockSpec(memory_space=pltpu.MemorySpace.HBM),
        scratch_shapes=[
            pltpu.VMEM((m, n), jnp.float32),
            pltpu.VMEM((m, n), jnp.bfloat16),
            pltpu.VMEM((m, n), jnp.int8),
            pltpu.VMEM((m, n), jnp.int8),
            pltpu.VMEM((SCALE_ROWS, n), jnp.float32),
            pltpu.VMEM((SCALE_ROWS, n), jnp.float32),
            pltpu.SemaphoreType.DMA((C,)),
            pltpu.SemaphoreType.DMA((C,)),
            pltpu.SemaphoreType.DMA((C,)),
            pltpu.SemaphoreType.DMA((C,)),
            pltpu.SemaphoreType.DMA((C,)),
            pltpu.SemaphoreType.DMA((C,)),
        ],
        compiler_params=pltpu.CompilerParams(collective_id=0),
    )(x)


# device time: 12704 ns/iter; 1.3756x vs baseline; 1.0519x over previous
import jax
import jax.numpy as jnp
from jax import lax
from jax.experimental import pallas as pl
from jax.experimental.pallas import tpu as pltpu

ROWS = (256, 256, 256, 192, 64)
C = len(ROWS)
OFFS = tuple(sum(ROWS[:i]) for i in range(C))
SCALE_ROWS = 8


def kernel(x):
    m, n = x.shape

    def body(
        x_ref, out_ref, qsend, qrecv, ssend, srecv,
        ss_send, ss_recv, d_send, d_recv,
    ):
        my_x = lax.axis_index("x")
        my_y = lax.axis_index("y")
        my_z = lax.axis_index("z")
        z_peer = (my_x, my_y, 1 - my_z)

        barrier_sem = pltpu.get_barrier_semaphore()
        pl.semaphore_signal(
            barrier_sem, inc=1, device_id=z_peer,
            device_id_type=pl.DeviceIdType.MESH,
        )

        def quantize(c):
            sl = pl.ds(OFFS[c], ROWS[c])
            s = jnp.max(jnp.abs(x_ref[sl, :])) / 127.0
            s = jnp.maximum(s, 1e-30)
            ssend[c, :] = jnp.full((n,), s, jnp.float32)
            qsend[sl, :] = jnp.clip(
                jnp.round(x_ref[sl, :] * (1.0 / s)), -127.0, 127.0
            ).astype(jnp.int8)

        def send(c):
            sl = pl.ds(OFFS[c], ROWS[c])
            sr = pltpu.make_async_remote_copy(
                src_ref=ssend.at[pl.ds(c, 1)],
                dst_ref=srecv.at[pl.ds(c, 1)],
                send_sem=ss_send.at[c],
                recv_sem=ss_recv.at[c],
                device_id=z_peer,
                device_id_type=pl.DeviceIdType.MESH,
            )
            sr.start()
            dr = pltpu.make_async_remote_copy(
                src_ref=qsend.at[sl],
                dst_ref=qrecv.at[sl],
                send_sem=d_send.at[c],
                recv_sem=d_recv.at[c],
                device_id=z_peer,
                device_id_type=pl.DeviceIdType.MESH,
            )
            dr.start()
            return sr, dr

        quantize(0)
        pl.semaphore_wait(barrier_sem, 1)
        rdmas = [send(0)]
        for c in range(1, C):
            quantize(c)
            rdmas.append(send(c))

        for c in range(C):
            sl = pl.ds(OFFS[c], ROWS[c])
            sr, dr = rdmas[c]
            sr.wait_recv()
            dr.wait_recv()
            peer_scale = srecv[c, 0]
            out_ref[sl, :] = (
                x_ref[sl, :] + qrecv[sl, :].astype(jnp.float32) * peer_scale
            ).astype(jnp.bfloat16)

        for c in range(C):
            rdmas[c][0].wait_send()
            rdmas[c][1].wait_send()

    return pl.pallas_call(
        body,
        out_shape=jax.ShapeDtypeStruct((m, n), jnp.bfloat16),
        in_specs=[pl.BlockSpec(memory_space=pltpu.VMEM)],
        out_specs=pl.BlockSpec(memory_space=pltpu.VMEM),
        scratch_shapes=[
            pltpu.VMEM((m, n), jnp.int8),
            pltpu.VMEM((m, n), jnp.int8),
            pltpu.VMEM((SCALE_ROWS, n), jnp.float32),
            pltpu.VMEM((SCALE_ROWS, n), jnp.float32),
            pltpu.SemaphoreType.DMA((C,)),
            pltpu.SemaphoreType.DMA((C,)),
            pltpu.SemaphoreType.DMA((C,)),
            pltpu.SemaphoreType.DMA((C,)),
        ],
        compiler_params=pltpu.CompilerParams(collective_id=0),
    )(x)
